# baseline (speedup 1.0000x reference)
import functools

import jax
import jax.numpy as jnp
import numpy as np
from jax import lax
from jax.experimental import pallas as pl
from jax.experimental.pallas import tpu as pltpu
from jax.experimental.pallas import tpu_sc as plsc

_NC = 2
_NS = 16
_L = 16
_NW = _NC * _NS


def _perm(D):
    p = []
    for g in range(D // 32):
        for k in range(16):
            p.extend((32 * g + k, 32 * g + 16 + k))
    return np.asarray(p, np.int32)


def _edge_lin(attr_bf, Wt_bf, b, block):
    M, K = attr_bf.shape
    Dout = Wt_bf.shape[1]
    b2 = b.reshape(1, Dout)

    def body(x_ref, w_ref, b_ref, o_ref):
        y = jnp.dot(x_ref[...], w_ref[...], preferred_element_type=jnp.float32)
        o_ref[...] = (y + b_ref[...]).astype(jnp.bfloat16)

    return pl.pallas_call(
        body,
        grid=(M // block,),
        in_specs=[
            pl.BlockSpec((block, K), lambda i: (i, 0)),
            pl.BlockSpec((K, Dout), lambda i: (0, 0)),
            pl.BlockSpec((1, Dout), lambda i: (0, 0)),
        ],
        out_specs=pl.BlockSpec((block, Dout), lambda i: (i, 0)),
        out_shape=jax.ShapeDtypeStruct((M, Dout), jnp.bfloat16),
    )(attr_bf, Wt_bf, b2)


def _mm_bias(x, Wt, b, relu, block):
    M, K = x.shape
    Dout = Wt.shape[1]
    b2 = b.reshape(1, Dout)

    def body(x_ref, w_ref, b_ref, o_ref):
        y = jnp.dot(x_ref[...], w_ref[...], preferred_element_type=jnp.float32, precision=lax.Precision.HIGHEST)
        y = y + b_ref[...]
        if relu:
            y = jnp.maximum(y, 0.0)
        o_ref[...] = y

    return pl.pallas_call(
        body,
        grid=(M // block,),
        in_specs=[
            pl.BlockSpec((block, K), lambda i: (i, 0)),
            pl.BlockSpec((K, Dout), lambda i: (0, 0)),
            pl.BlockSpec((1, Dout), lambda i: (0, 0)),
        ],
        out_specs=pl.BlockSpec((block, Dout), lambda i: (i, 0)),
        out_shape=jax.ShapeDtypeStruct((M, Dout), jnp.float32),
    )(x, Wt, b2)


def _node_update(h, P, svec, W1t, b1, W2t, b2, block):
    M, D = h.shape
    b1r = b1.reshape(1, D)
    b2r = b2.reshape(1, D)

    def body(h_ref, p_ref, s_ref, w1_ref, b1_ref, w2_ref, b2_ref, o_ref):
        hn = h_ref[...] * s_ref[...] + p_ref[0] + p_ref[1]
        t = jnp.dot(hn, w1_ref[...], preferred_element_type=jnp.float32, precision=lax.Precision.HIGHEST)
        t = jnp.maximum(t + b1_ref[...], 0.0)
        y = jnp.dot(t, w2_ref[...], preferred_element_type=jnp.float32, precision=lax.Precision.HIGHEST)
        o_ref[...] = jnp.maximum(y + b2_ref[...], 0.0)

    return pl.pallas_call(
        body,
        grid=(M // block,),
        in_specs=[
            pl.BlockSpec((block, D), lambda i: (i, 0)),
            pl.BlockSpec((2, block, D), lambda i: (0, i, 0)),
            pl.BlockSpec((1, D), lambda i: (0, 0)),
            pl.BlockSpec((D, D), lambda i: (0, 0)),
            pl.BlockSpec((1, D), lambda i: (0, 0)),
            pl.BlockSpec((D, D), lambda i: (0, 0)),
            pl.BlockSpec((1, D), lambda i: (0, 0)),
        ],
        out_specs=pl.BlockSpec((block, D), lambda i: (i, 0)),
        out_shape=jax.ShapeDtypeStruct((M, D), jnp.float32),
    )(h, P, svec, W1t, b1r, W2t, b2r)


@functools.lru_cache(maxsize=None)
def _make_sc_aggr(N, D, E, K):
    NP = ((N + _NS * K - 1) // (_NS * K)) * (_NS * K)
    per_w = E // _NW
    steps = per_w // K
    assert steps % 2 == 0
    rows_t = NP // _NS
    assert rows_t % K == 0
    mesh = plsc.VectorSubcoreMesh(core_axis_name="c", subcore_axis_name="s")

    @functools.partial(
        pl.kernel,
        out_type=jax.ShapeDtypeStruct((_NC, NP, D), jnp.float32),
        mesh=mesh,
        scratch_types=[
            pltpu.VMEM((1, K), jnp.int32),
            pltpu.VMEM((1, K), jnp.int32),
            pltpu.VMEM((1, K), jnp.int32),
            pltpu.VMEM((1, K), jnp.int32),
            pltpu.VMEM((K, D), jnp.float32),
            pltpu.VMEM((K, D), jnp.float32),
            pltpu.VMEM((K, D // 2), jnp.int32),
            pltpu.VMEM((K, D // 2), jnp.int32),
            pltpu.VMEM((K, D), jnp.float32),
            pltpu.VMEM_SHARED((NP, D), jnp.float32),
            pltpu.SemaphoreType.DMA,
            pltpu.SemaphoreType.DMA,
            pltpu.SemaphoreType.DMA,
            pltpu.SemaphoreType.DMA,
            pltpu.SemaphoreType.DMA,
            pltpu.SemaphoreType.DMA,
            pltpu.SemaphoreType.DMA,
            pltpu.SemaphoreType.DMA,
        ],
    )
    def sc_aggr(h_hbm, e_hbm, src_hbm, dst_hbm, out_hbm,
                src0, src1, dst0, dst1, g0, g1, eb0, eb1, mbuf, aggr_sh,
                sg0, sg1, se0, se1, si0, si1, sd0, sd1):
        c = lax.axis_index("c")
        s = lax.axis_index("s")
        wid = c * _NS + s
        last = steps - 1

        def clamp(j):
            return jnp.minimum(j, last)

        def issue_src(j, buf, sem):
            pltpu.async_copy(src_hbm.at[wid, clamp(j)], buf, sem)

        def issue_dst(j, buf, sem):
            pltpu.async_copy(dst_hbm.at[wid, clamp(j)], buf, sem)

        def wait_idx(buf, sem):
            pltpu.make_async_copy(src_hbm.at[wid, 0], buf, sem).wait()

        def issue_gather(sbuf, gbuf, sem):
            pltpu.async_copy(h_hbm.at[sbuf.at[0]], gbuf, sem)

        def wait_rows(src, dst, sem):
            pltpu.make_async_copy(src, dst, sem).wait()

        def issue_eload(j, buf, sem):
            pltpu.async_copy(e_hbm.at[wid, clamp(j)], buf, sem)

        hi_mask = jnp.full((_L,), -65536, jnp.int32)

        def compute(gbuf, ebuf):
            def row(r, rc):
                for g in range(D // (2 * _L)):
                    ew = ebuf[r, pl.ds(_L * g, _L)]
                    ea = lax.bitcast_convert_type(ew << 16, jnp.float32)
                    eb = lax.bitcast_convert_type(ew & hi_mask, jnp.float32)
                    for half, ev in ((0, ea), (1, eb)):
                        sl = pl.ds(2 * _L * g + half * _L, _L)
                        mbuf[r, sl] = jnp.maximum(
                            gbuf[r, sl] + ev, 0.0)
                return rc
            lax.fori_loop(0, K, row, 0)

        zero = jnp.zeros((_L,), jnp.float32)
        for r in range(K):
            for j in range(D // _L):
                mbuf[r, pl.ds(j * _L, _L)] = zero

        def zstep(i, carry):
            pltpu.sync_copy(mbuf, aggr_sh.at[pl.ds(s * rows_t + i * K, K)])
            return carry
        lax.fori_loop(0, rows_t // K, zstep, 0)

        plsc.subcore_barrier()

        issue_src(0, src0, si0)
        issue_dst(0, dst0, sd0)
        issue_src(1, src1, si1)
        issue_dst(1, dst1, sd1)
        wait_idx(src0, si0)
        issue_gather(src0, g0, sg0)
        issue_eload(0, eb0, se0)

        def body(t, carry):
            i = 2 * t
            wait_idx(src1, si1)
            issue_gather(src1, g1, sg1)
            issue_eload(i + 1, eb1, se1)
            wait_rows(h_hbm.at[src0.at[0]], g0, sg0)
            wait_rows(e_hbm.at[wid, 0], eb0, se0)
            issue_src(i + 2, src0, si0)
            compute(g0, eb0)
            wait_idx(dst0, sd0)
            pltpu.sync_copy(mbuf, aggr_sh.at[dst0.at[0]], add=True)
            issue_dst(i + 2, dst0, sd0)

            wait_idx(src0, si0)
            issue_gather(src0, g0, sg0)
            issue_eload(i + 2, eb0, se0)
            wait_rows(h_hbm.at[src1.at[0]], g1, sg1)
            wait_rows(e_hbm.at[wid, 0], eb1, se1)
            issue_src(i + 3, src1, si1)
            compute(g1, eb1)
            wait_idx(dst1, sd1)
            pltpu.sync_copy(mbuf, aggr_sh.at[dst1.at[0]], add=True)
            issue_dst(i + 3, dst1, sd1)
            return carry
        lax.fori_loop(0, steps // 2, body, 0)

        wait_rows(h_hbm.at[src0.at[0]], g0, sg0)
        wait_rows(e_hbm.at[wid, 0], eb0, se0)
        wait_idx(src1, si1)
        wait_idx(dst0, sd0)
        wait_idx(dst1, sd1)

        plsc.subcore_barrier()

        pltpu.sync_copy(aggr_sh.at[pl.ds(s * rows_t, rows_t)],
                        out_hbm.at[c, pl.ds(s * rows_t, rows_t)])

    return sc_aggr


def _sc_aggr_call(h, e4, src_r, dst_r, K, E):
    N, D = h.shape
    return _make_sc_aggr(N, D, E, K)(h, e4, src_r, dst_r)


def kernel(x, edge_index, edge_attr, W0, b0, We1, be1, eps1, W11, b11,
           W12, b12, We2, be2, eps2, W21, b21, W22, b22):
    N, D = x.shape
    E = edge_index.shape[1]
    K = 40
    per_w = E // _NW
    steps = per_w // K
    perm = _perm(D)

    src_r = edge_index[0].reshape(_NW, steps, 1, K)
    dst_r = edge_index[1].reshape(_NW, steps, 1, K)
    attr_bf = edge_attr.astype(jnp.bfloat16)

    h0 = _mm_bias(x, W0.T, b0, relu=True, block=1000)

    def to_words(e):
        ew = jax.lax.bitcast_convert_type(e.reshape(E, D // 2, 2), jnp.int32)
        return ew.reshape(_NW, steps, K, D // 2)

    e1 = to_words(_edge_lin(attr_bf, We1.T[:, perm].astype(jnp.bfloat16),
                            be1[perm], block=16000))
    P1 = _sc_aggr_call(h0, e1, src_r, dst_r, K, E)
    s1 = jnp.full((1, D), 1.0 + eps1, jnp.float32)
    h1 = _node_update(h0, P1, s1, W11.T, b11, W12.T, b12, block=1000)

    e2 = to_words(_edge_lin(attr_bf, We2.T[:, perm].astype(jnp.bfloat16),
                            be2[perm], block=16000))
    P2 = _sc_aggr_call(h1, e2, src_r, dst_r, K, E)
    s2 = jnp.full((1, D), 1.0 + eps2, jnp.float32)
    out = _node_update(h1, P2, s2, W21.T, b21, W22.T, b22, block=1000)

    return out


# speedup vs baseline: 2.8412x; 2.8412x over previous
import functools

import jax
import jax.numpy as jnp
from jax import lax
from jax.experimental import pallas as pl
from jax.experimental.pallas import tpu as pltpu
from jax.experimental.pallas import tpu_sc as plsc

_NC = 2
_NS = 16
_L = 16
_NW = _NC * _NS


def _edge_lin(attr_bf, Wt_bf, b, block):
    M, K = attr_bf.shape
    Dout = Wt_bf.shape[1]
    b2 = b.reshape(1, Dout)

    def body(x_ref, w_ref, b_ref, o_ref):
        y = jnp.dot(x_ref[...], w_ref[...], preferred_element_type=jnp.float32)
        o_ref[...] = y + b_ref[...]

    return pl.pallas_call(
        body,
        grid=(M // block,),
        in_specs=[
            pl.BlockSpec((block, K), lambda i: (i, 0)),
            pl.BlockSpec((K, Dout), lambda i: (0, 0)),
            pl.BlockSpec((1, Dout), lambda i: (0, 0)),
        ],
        out_specs=pl.BlockSpec((block, Dout), lambda i: (i, 0)),
        out_shape=jax.ShapeDtypeStruct((M, Dout), jnp.float32),
    )(attr_bf, Wt_bf, b2)


def _mm_bias(x, Wt, b, relu, block):
    M, K = x.shape
    Dout = Wt.shape[1]
    b2 = b.reshape(1, Dout)

    def body(x_ref, w_ref, b_ref, o_ref):
        y = jnp.dot(x_ref[...], w_ref[...], preferred_element_type=jnp.float32, precision=lax.Precision.HIGHEST)
        y = y + b_ref[...]
        if relu:
            y = jnp.maximum(y, 0.0)
        o_ref[...] = y

    return pl.pallas_call(
        body,
        grid=(M // block,),
        in_specs=[
            pl.BlockSpec((block, K), lambda i: (i, 0)),
            pl.BlockSpec((K, Dout), lambda i: (0, 0)),
            pl.BlockSpec((1, Dout), lambda i: (0, 0)),
        ],
        out_specs=pl.BlockSpec((block, Dout), lambda i: (i, 0)),
        out_shape=jax.ShapeDtypeStruct((M, Dout), jnp.float32),
    )(x, Wt, b2)


def _node_update(h, P, svec, W1t, b1, W2t, b2, block):
    M, D = h.shape
    b1r = b1.reshape(1, D)
    b2r = b2.reshape(1, D)

    def body(h_ref, p_ref, s_ref, w1_ref, b1_ref, w2_ref, b2_ref, o_ref):
        hn = h_ref[...] * s_ref[...] + p_ref[0] + p_ref[1]
        t = jnp.dot(hn, w1_ref[...], preferred_element_type=jnp.float32, precision=lax.Precision.HIGHEST)
        t = jnp.maximum(t + b1_ref[...], 0.0)
        y = jnp.dot(t, w2_ref[...], preferred_element_type=jnp.float32, precision=lax.Precision.HIGHEST)
        o_ref[...] = jnp.maximum(y + b2_ref[...], 0.0)

    return pl.pallas_call(
        body,
        grid=(M // block,),
        in_specs=[
            pl.BlockSpec((block, D), lambda i: (i, 0)),
            pl.BlockSpec((2, block, D), lambda i: (0, i, 0)),
            pl.BlockSpec((1, D), lambda i: (0, 0)),
            pl.BlockSpec((D, D), lambda i: (0, 0)),
            pl.BlockSpec((1, D), lambda i: (0, 0)),
            pl.BlockSpec((D, D), lambda i: (0, 0)),
            pl.BlockSpec((1, D), lambda i: (0, 0)),
        ],
        out_specs=pl.BlockSpec((block, D), lambda i: (i, 0)),
        out_shape=jax.ShapeDtypeStruct((M, D), jnp.float32),
    )(h, P, svec, W1t, b1r, W2t, b2r)


@functools.lru_cache(maxsize=None)
def _make_sc_aggr(N, D, E, K):
    NP = ((N + _NS * K - 1) // (_NS * K)) * (_NS * K)
    per_w = E // _NW
    steps = per_w // K
    assert steps % 2 == 0
    rows_t = NP // _NS
    assert rows_t % K == 0
    mesh = plsc.VectorSubcoreMesh(core_axis_name="c", subcore_axis_name="s")

    @functools.partial(
        pl.kernel,
        out_type=jax.ShapeDtypeStruct((_NC, NP, D), jnp.float32),
        mesh=mesh,
        scratch_types=[
            pltpu.VMEM((1, K), jnp.int32),
            pltpu.VMEM((1, K), jnp.int32),
            pltpu.VMEM((1, K), jnp.int32),
            pltpu.VMEM((1, K), jnp.int32),
            pltpu.VMEM((K, D), jnp.float32),
            pltpu.VMEM((K, D), jnp.float32),
            pltpu.VMEM((K, D), jnp.float32),
            pltpu.VMEM((K, D), jnp.float32),
            pltpu.VMEM_SHARED((NP, D), jnp.float32),
            pltpu.SemaphoreType.DMA,
            pltpu.SemaphoreType.DMA,
            pltpu.SemaphoreType.DMA,
            pltpu.SemaphoreType.DMA,
            pltpu.SemaphoreType.DMA,
            pltpu.SemaphoreType.DMA,
            pltpu.SemaphoreType.DMA,
            pltpu.SemaphoreType.DMA,
            pltpu.SemaphoreType.DMA,
            pltpu.SemaphoreType.DMA,
        ],
    )
    def sc_aggr(h_hbm, e_hbm, src_hbm, dst_hbm, out_hbm,
                src0, src1, dst0, dst1, g0, g1, eb0, eb1, aggr_sh,
                sg0, sg1, se0, se1, si0, si1, sd0, sd1, ss0, ss1):
        c = lax.axis_index("c")
        s = lax.axis_index("s")
        wid = c * _NS + s
        last = steps - 1

        def clamp(j):
            return jnp.minimum(j, last)

        def issue_src(j, buf, sem):
            pltpu.async_copy(src_hbm.at[wid, clamp(j)], buf, sem)

        def issue_dst(j, buf, sem):
            pltpu.async_copy(dst_hbm.at[wid, clamp(j)], buf, sem)

        def wait_idx(buf, sem):
            pltpu.make_async_copy(src_hbm.at[wid, 0], buf, sem).wait()

        def issue_gather(sbuf, gbuf, sem):
            pltpu.async_copy(h_hbm.at[sbuf.at[0]], gbuf, sem)

        def wait_rows(dst, sem):
            pltpu.make_async_copy(e_hbm.at[pl.ds(0, K)], dst, sem).wait()

        def issue_eload(j, buf, sem):
            eb = wid * per_w + clamp(j) * K
            pltpu.async_copy(e_hbm.at[pl.ds(eb, K)], buf, sem)

        def compute(gbuf, mbuf):
            def row(r, rc):
                for rr in range(2):
                    for j in range(D // _L):
                        sl = pl.ds(j * _L, _L)
                        mbuf[2 * r + rr, sl] = jnp.maximum(
                            gbuf[2 * r + rr, sl] + mbuf[2 * r + rr, sl], 0.0)
                return rc
            lax.fori_loop(0, K // 2, row, 0)

        zero = jnp.zeros((_L,), jnp.float32)
        for r in range(K):
            for j in range(D // _L):
                eb0[r, pl.ds(j * _L, _L)] = zero

        def zstep(i, carry):
            pltpu.sync_copy(eb0, aggr_sh.at[pl.ds(s * rows_t + i * K, K)])
            return carry
        lax.fori_loop(0, rows_t // K, zstep, 0)

        plsc.subcore_barrier()

        def issue_scatter(mbuf, dbuf, sem):
            pltpu.async_copy(mbuf, aggr_sh.at[dbuf.at[0]], sem, add=True)

        def wait_scatter(mbuf, dbuf, sem):
            pltpu.make_async_copy(mbuf, aggr_sh.at[dbuf.at[0]], sem).wait()

        issue_src(0, src0, si0)
        issue_dst(0, dst0, sd0)
        issue_src(1, src1, si1)
        wait_idx(src0, si0)
        issue_gather(src0, g0, sg0)
        issue_eload(0, eb0, se0)

        def body(t, carry):
            i = 2 * t
            @pl.when(t > 0)
            def _():
                wait_scatter(eb1, dst1, ss1)
            issue_dst(i + 1, dst1, sd1)
            wait_idx(src1, si1)
            issue_gather(src1, g1, sg1)
            issue_eload(i + 1, eb1, se1)
            wait_rows(g0, sg0)
            wait_rows(eb0, se0)
            issue_src(i + 2, src0, si0)
            compute(g0, eb0)
            wait_idx(dst0, sd0)
            issue_scatter(eb0, dst0, ss0)

            wait_idx(src0, si0)
            issue_gather(src0, g0, sg0)
            wait_rows(g1, sg1)
            wait_rows(eb1, se1)
            wait_scatter(eb0, dst0, ss0)
            issue_dst(i + 2, dst0, sd0)
            issue_eload(i + 2, eb0, se0)
            issue_src(i + 3, src1, si1)
            compute(g1, eb1)
            wait_idx(dst1, sd1)
            issue_scatter(eb1, dst1, ss1)
            return carry
        lax.fori_loop(0, steps // 2, body, 0)

        wait_scatter(eb1, dst1, ss1)
        wait_rows(g0, sg0)
        wait_rows(eb0, se0)
        wait_idx(src1, si1)
        wait_idx(dst0, sd0)

        plsc.subcore_barrier()

        pltpu.sync_copy(aggr_sh.at[pl.ds(s * rows_t, rows_t)],
                        out_hbm.at[c, pl.ds(s * rows_t, rows_t)])

    return sc_aggr


def _sc_aggr_call(h, e, src_r, dst_r, K):
    N, D = h.shape
    E = e.shape[0]
    return _make_sc_aggr(N, D, E, K)(h, e, src_r, dst_r)


def kernel(x, edge_index, edge_attr, W0, b0, We1, be1, eps1, W11, b11,
           W12, b12, We2, be2, eps2, W21, b21, W22, b22):
    N, D = x.shape
    E = edge_index.shape[1]
    K = 40

    per_w = E // _NW
    src_r = edge_index[0].reshape(_NW, per_w // K, 1, K)
    dst_r = edge_index[1].reshape(_NW, per_w // K, 1, K)

    attr_bf = edge_attr.astype(jnp.bfloat16)
    h0 = _mm_bias(x, W0.T, b0, relu=True, block=1000)

    e1 = _edge_lin(attr_bf, We1.T.astype(jnp.bfloat16), be1, block=16000)
    P1 = _sc_aggr_call(h0, e1, src_r, dst_r, K)
    s1 = jnp.full((1, D), 1.0 + eps1, jnp.float32)
    h1 = _node_update(h0, P1, s1, W11.T, b11, W12.T, b12, block=1000)

    e2 = _edge_lin(attr_bf, We2.T.astype(jnp.bfloat16), be2, block=16000)
    P2 = _sc_aggr_call(h1, e2, src_r, dst_r, K)
    s2 = jnp.full((1, D), 1.0 + eps2, jnp.float32)
    out = _node_update(h1, P2, s2, W21.T, b21, W22.T, b22, block=1000)

    return out


# speedup vs baseline: 2.9869x; 1.0513x over previous
import functools

import jax
import jax.numpy as jnp
from jax import lax
from jax.experimental import pallas as pl
from jax.experimental.pallas import tpu as pltpu
from jax.experimental.pallas import tpu_sc as plsc

_NC = 2
_NS = 16
_L = 16
_NW = _NC * _NS


def _edge_lin(attr_bf, Wt_bf, b, block):
    M, K = attr_bf.shape
    Dout = Wt_bf.shape[1]
    b2 = b.reshape(1, Dout)

    def body(x_ref, w_ref, b_ref, o_ref):
        y = jnp.dot(x_ref[...], w_ref[...], preferred_element_type=jnp.float32)
        o_ref[...] = y + b_ref[...]

    return pl.pallas_call(
        body,
        grid=(M // block,),
        in_specs=[
            pl.BlockSpec((block, K), lambda i: (i, 0)),
            pl.BlockSpec((K, Dout), lambda i: (0, 0)),
            pl.BlockSpec((1, Dout), lambda i: (0, 0)),
        ],
        out_specs=pl.BlockSpec((block, Dout), lambda i: (i, 0)),
        out_shape=jax.ShapeDtypeStruct((M, Dout), jnp.float32),
    )(attr_bf, Wt_bf, b2)


def _mm_bias(x, Wt, b, relu, block):
    M, K = x.shape
    Dout = Wt.shape[1]
    b2 = b.reshape(1, Dout)

    def body(x_ref, w_ref, b_ref, o_ref):
        y = jnp.dot(x_ref[...], w_ref[...], preferred_element_type=jnp.float32, precision=lax.Precision.HIGHEST)
        y = y + b_ref[...]
        if relu:
            y = jnp.maximum(y, 0.0)
        o_ref[...] = y

    return pl.pallas_call(
        body,
        grid=(M // block,),
        in_specs=[
            pl.BlockSpec((block, K), lambda i: (i, 0)),
            pl.BlockSpec((K, Dout), lambda i: (0, 0)),
            pl.BlockSpec((1, Dout), lambda i: (0, 0)),
        ],
        out_specs=pl.BlockSpec((block, Dout), lambda i: (i, 0)),
        out_shape=jax.ShapeDtypeStruct((M, Dout), jnp.float32),
    )(x, Wt, b2)


def _node_update(h, P, svec, W1t, b1, W2t, b2, block):
    M, D = h.shape
    b1r = b1.reshape(1, D)
    b2r = b2.reshape(1, D)

    def body(h_ref, p_ref, s_ref, w1_ref, b1_ref, w2_ref, b2_ref, o_ref):
        hn = h_ref[...] * s_ref[...] + p_ref[0] + p_ref[1]
        t = jnp.dot(hn, w1_ref[...], preferred_element_type=jnp.float32, precision=lax.Precision.HIGHEST)
        t = jnp.maximum(t + b1_ref[...], 0.0)
        y = jnp.dot(t, w2_ref[...], preferred_element_type=jnp.float32, precision=lax.Precision.HIGHEST)
        o_ref[...] = jnp.maximum(y + b2_ref[...], 0.0)

    return pl.pallas_call(
        body,
        grid=(M // block,),
        in_specs=[
            pl.BlockSpec((block, D), lambda i: (i, 0)),
            pl.BlockSpec((2, block, D), lambda i: (0, i, 0)),
            pl.BlockSpec((1, D), lambda i: (0, 0)),
            pl.BlockSpec((D, D), lambda i: (0, 0)),
            pl.BlockSpec((1, D), lambda i: (0, 0)),
            pl.BlockSpec((D, D), lambda i: (0, 0)),
            pl.BlockSpec((1, D), lambda i: (0, 0)),
        ],
        out_specs=pl.BlockSpec((block, D), lambda i: (i, 0)),
        out_shape=jax.ShapeDtypeStruct((M, D), jnp.float32),
    )(h, P, svec, W1t, b1r, W2t, b2r)


@functools.lru_cache(maxsize=None)
def _make_sc_aggr(N, D, E, K):
    NP = ((N + _NS * K - 1) // (_NS * K)) * (_NS * K)
    per_w = E // _NW
    steps = per_w // K
    assert steps % 2 == 0
    rows_t = NP // _NS
    assert rows_t % K == 0
    mesh = plsc.VectorSubcoreMesh(core_axis_name="c", subcore_axis_name="s")

    @functools.partial(
        pl.kernel,
        out_type=jax.ShapeDtypeStruct((_NC, NP, D), jnp.float32),
        mesh=mesh,
        scratch_types=[
            pltpu.VMEM((1, K), jnp.int32),
            pltpu.VMEM((1, K), jnp.int32),
            pltpu.VMEM((1, K), jnp.int32),
            pltpu.VMEM((1, K), jnp.int32),
            pltpu.VMEM((K, D), jnp.float32),
            pltpu.VMEM((K, D), jnp.float32),
            pltpu.VMEM((K, D), jnp.float32),
            pltpu.VMEM((K, D), jnp.float32),
            pltpu.VMEM_SHARED((NP, D), jnp.float32),
            pltpu.SemaphoreType.DMA,
            pltpu.SemaphoreType.DMA,
            pltpu.SemaphoreType.DMA,
            pltpu.SemaphoreType.DMA,
            pltpu.SemaphoreType.DMA,
            pltpu.SemaphoreType.DMA,
            pltpu.SemaphoreType.DMA,
            pltpu.SemaphoreType.DMA,
            pltpu.SemaphoreType.DMA,
            pltpu.SemaphoreType.DMA,
        ],
    )
    def sc_aggr(h_hbm, e_hbm, src_hbm, dst_hbm, out_hbm,
                src0, src1, dst0, dst1, g0, g1, eb0, eb1, aggr_sh,
                sg0, sg1, se0, se1, si0, si1, sd0, sd1, ss0, ss1):
        c = lax.axis_index("c")
        s = lax.axis_index("s")
        wid = c * _NS + s
        last = steps - 1

        def clamp(j):
            return jnp.minimum(j, last)

        def issue_src(j, buf, sem):
            pltpu.async_copy(src_hbm.at[wid, clamp(j)], buf, sem)

        def issue_dst(j, buf, sem):
            pltpu.async_copy(dst_hbm.at[wid, clamp(j)], buf, sem)

        def wait_idx(buf, sem):
            pltpu.make_async_copy(src_hbm.at[wid, 0], buf, sem).wait()

        def issue_gather(sbuf, gbuf, sem):
            pltpu.async_copy(h_hbm.at[sbuf.at[0]], gbuf, sem)

        def wait_rows(dst, sem):
            pltpu.make_async_copy(e_hbm.at[pl.ds(0, K)], dst, sem).wait()

        def issue_eload(j, buf, sem):
            eb = wid * per_w + clamp(j) * K
            pltpu.async_copy(e_hbm.at[pl.ds(eb, K)], buf, sem)

        def compute(gbuf, mbuf):
            def row(r, rc):
                for rr in range(2):
                    for j in range(D // _L):
                        sl = pl.ds(j * _L, _L)
                        mbuf[2 * r + rr, sl] = jnp.maximum(
                            gbuf[2 * r + rr, sl] + mbuf[2 * r + rr, sl], 0.0)
                return rc
            lax.fori_loop(0, K // 2, row, 0)

        zero = jnp.zeros((_L,), jnp.float32)
        for r in range(K):
            for j in range(D // _L):
                eb0[r, pl.ds(j * _L, _L)] = zero

        def zstep(i, carry):
            pltpu.sync_copy(eb0, aggr_sh.at[pl.ds(s * rows_t + i * K, K)])
            return carry
        lax.fori_loop(0, rows_t // K, zstep, 0)

        plsc.subcore_barrier()

        def issue_scatter(mbuf, dbuf, sem):
            pltpu.async_copy(mbuf, aggr_sh.at[dbuf.at[0]], sem, add=True)

        def wait_scatter(mbuf, dbuf, sem):
            pltpu.make_async_copy(mbuf, aggr_sh.at[dbuf.at[0]], sem).wait()

        issue_src(0, src0, si0)
        issue_dst(0, dst0, sd0)
        issue_src(1, src1, si1)
        wait_idx(src0, si0)
        issue_gather(src0, g0, sg0)
        issue_eload(0, eb0, se0)

        def body(t, carry):
            i = 2 * t
            @pl.when(t > 0)
            def _():
                wait_scatter(eb1, dst1, ss1)
            issue_dst(i + 1, dst1, sd1)
            wait_idx(src1, si1)
            issue_gather(src1, g1, sg1)
            issue_eload(i + 1, eb1, se1)
            wait_rows(g0, sg0)
            wait_rows(eb0, se0)
            issue_src(i + 2, src0, si0)
            compute(g0, eb0)
            wait_idx(dst0, sd0)
            issue_scatter(eb0, dst0, ss0)

            wait_idx(src0, si0)
            issue_gather(src0, g0, sg0)
            wait_rows(g1, sg1)
            wait_rows(eb1, se1)
            wait_scatter(eb0, dst0, ss0)
            issue_dst(i + 2, dst0, sd0)
            issue_eload(i + 2, eb0, se0)
            issue_src(i + 3, src1, si1)
            compute(g1, eb1)
            wait_idx(dst1, sd1)
            issue_scatter(eb1, dst1, ss1)
            return carry
        lax.fori_loop(0, steps // 2, body, 0)

        wait_scatter(eb1, dst1, ss1)
        wait_rows(g0, sg0)
        wait_rows(eb0, se0)
        wait_idx(src1, si1)
        wait_idx(dst0, sd0)

        plsc.subcore_barrier()

        pltpu.sync_copy(aggr_sh.at[pl.ds(s * rows_t, rows_t)],
                        out_hbm.at[c, pl.ds(s * rows_t, rows_t)])

    return sc_aggr


def _sc_aggr_call(h, e, src_r, dst_r, K):
    N, D = h.shape
    E = e.shape[0]
    return _make_sc_aggr(N, D, E, K)(h, e, src_r, dst_r)


def kernel(x, edge_index, edge_attr, W0, b0, We1, be1, eps1, W11, b11,
           W12, b12, We2, be2, eps2, W21, b21, W22, b22):
    N, D = x.shape
    E = edge_index.shape[1]
    K = 40

    per_w = E // _NW
    src_r = edge_index[0].reshape(_NW, per_w // K, 1, K)
    dst_r = edge_index[1].reshape(_NW, per_w // K, 1, K)

    attr_bf = edge_attr.astype(jnp.bfloat16)
    h0 = _mm_bias(x, W0.T, b0, relu=True, block=2000)

    e1 = _edge_lin(attr_bf, We1.T.astype(jnp.bfloat16), be1, block=32000)
    P1 = _sc_aggr_call(h0, e1, src_r, dst_r, K)
    s1 = jnp.full((1, D), 1.0 + eps1, jnp.float32)
    h1 = _node_update(h0, P1, s1, W11.T, b11, W12.T, b12, block=2000)

    e2 = _edge_lin(attr_bf, We2.T.astype(jnp.bfloat16), be2, block=32000)
    P2 = _sc_aggr_call(h1, e2, src_r, dst_r, K)
    s2 = jnp.full((1, D), 1.0 + eps2, jnp.float32)
    out = _node_update(h1, P2, s2, W21.T, b21, W22.T, b22, block=2000)

    return out
